# baseline (speedup 1.0000x reference)
import functools

import jax
import jax.numpy as jnp
from jax import lax
from jax.experimental import pallas as pl
from jax.experimental.pallas import tpu as pltpu
from jax.experimental.pallas import tpu_sc as plsc

NBLK = 4
N = 10000
E = 65536
NG = 64
D = 1024
CK = 128
NCHUNK = D // CK
RT = 1000
NRT = N // RT
NSC = 2
NSUB = 16
EG = 128
EGRP = E // NSUB // EG
DEG_W = 128
DEG_GRP = E // (NSC * NSUB) // EG
NP = 10240
RPT = NP // NSUB
D2 = D // 2
NCH = NCHUNK // 2


def _deg_body(dst_hbm, ones_hbm, zeros_hbm, out_hbm, ones_v, idx_v, acc_sh):
    core = lax.axis_index("c")
    sub = lax.axis_index("s")
    pltpu.sync_copy(ones_hbm, ones_v)
    pltpu.sync_copy(zeros_hbm, acc_sh.at[pl.ds(sub * RPT, RPT)])
    plsc.subcore_barrier()
    base = (core * NSUB + sub) * (E // (NSC * NSUB))

    def body(g, carry):
        pltpu.sync_copy(dst_hbm.at[pl.ds(base + g * EG, EG)], idx_v)
        pltpu.sync_copy(ones_v, acc_sh.at[idx_v], add=True)
        return carry

    lax.fori_loop(0, DEG_GRP, body, 0)
    plsc.subcore_barrier()
    pltpu.sync_copy(acc_sh.at[pl.ds(sub * RPT, RPT)],
                    out_hbm.at[core, pl.ds(sub * RPT, RPT)])


@functools.cache
def _sc_kernels():
    mesh = plsc.VectorSubcoreMesh(core_axis_name="c", subcore_axis_name="s")
    deg = pl.kernel(
        _deg_body,
        out_type=jax.ShapeDtypeStruct((NSC, NP, DEG_W), jnp.float32),
        mesh=mesh,
        scratch_types=[
            pltpu.VMEM((EG, DEG_W), jnp.float32),
            pltpu.VMEM((EG,), jnp.int32),
            pltpu.VMEM_SHARED((NP, DEG_W), jnp.float32),
        ],
    )
    edge = pl.kernel(
        _edge_body,
        out_type=jax.ShapeDtypeStruct((NCH, NP, CK), jnp.float32),
        mesh=mesh,
        scratch_types=[
            pltpu.VMEM((EG, CK), jnp.float32),
            pltpu.VMEM((EG, CK), jnp.float32),
            pltpu.VMEM((EGRP, EG), jnp.int32),
            pltpu.VMEM((EGRP, EG), jnp.int32),
            pltpu.VMEM((EGRP, EG), jnp.int32),
            pltpu.VMEM_SHARED((NP, CK), jnp.float32),
            pltpu.SemaphoreType.DMA,
            pltpu.SemaphoreType.DMA,
        ],
    )
    return deg, edge


def _edge_body(xws_hbm, src_hbm, dst_hbm, acc_hbm,
               rows0_v, rows1_v, idxs_v, idxo_v, idxd_v, acc_sh, sem0, sem1):
    core = lax.axis_index("c")
    sub = lax.axis_index("s")
    pltpu.sync_copy(src_hbm.at[sub], idxs_v)
    pltpu.sync_copy(dst_hbm.at[sub], idxd_v)

    def chunk(cc, carry):
        chunk_id = core * (NCH // NSC) + cc
        off = chunk_id * NP

        def obody(g, inner):
            for j in range(EG // 16):
                sl = pl.ds(j * 16, 16)
                idxo_v[g, sl] = idxs_v[g, sl] + off
            return inner

        lax.fori_loop(0, EGRP, obody, 0)
        pltpu.sync_copy(xws_hbm.at[pl.ds(off + sub * RPT, RPT)],
                        acc_sh.at[pl.ds(sub * RPT, RPT)])
        plsc.subcore_barrier()

        pltpu.async_copy(xws_hbm.at[idxo_v.at[0]], rows0_v, sem0)

        def g_body(p, inner):
            g0 = 2 * p
            pltpu.async_copy(xws_hbm.at[idxo_v.at[g0 + 1]], rows1_v, sem1)
            pltpu.make_async_copy(xws_hbm.at[idxo_v.at[g0]],
                                  rows0_v, sem0).wait()
            pltpu.sync_copy(rows0_v, acc_sh.at[idxd_v.at[g0]], add=True)

            @pl.when(p + 1 < EGRP // 2)
            def _():
                pltpu.async_copy(xws_hbm.at[idxo_v.at[g0 + 2]], rows0_v, sem0)

            pltpu.make_async_copy(xws_hbm.at[idxo_v.at[g0 + 1]],
                                  rows1_v, sem1).wait()
            pltpu.sync_copy(rows1_v, acc_sh.at[idxd_v.at[g0 + 1]], add=True)
            return inner

        lax.fori_loop(0, EGRP // 2, g_body, 0)
        plsc.subcore_barrier()
        pltpu.sync_copy(acc_sh.at[pl.ds(sub * RPT, RPT)],
                        acc_hbm.at[chunk_id, pl.ds(sub * RPT, RPT)])
        plsc.subcore_barrier()
        return carry

    lax.fori_loop(0, NCH // NSC, chunk, 0)


def _pre_body(degp_ref, dis_ref):
    deg = degp_ref[0, :, 0:1] + degp_ref[1, :, 0:1] + 1.0
    dis_ref[...] = lax.rsqrt(deg)


_pre_kernel = pl.pallas_call(
    _pre_body,
    grid=(NRT,),
    in_specs=[
        pl.BlockSpec((NSC, RT, DEG_W), lambda m: (0, m, 0)),
    ],
    out_specs=pl.BlockSpec((RT, 1), lambda m: (m, 0)),
    out_shape=jax.ShapeDtypeStruct((N, 1), jnp.float32),
)


def _mm_body(ha_ref, hb_ref, dis_ref, w_ref, out_ref):
    dis = dis_ref[...]
    acc = jnp.dot(ha_ref[...] * dis, w_ref[:D2, :],
                  preferred_element_type=jnp.float32) \
        + jnp.dot(hb_ref[...] * dis, w_ref[D2:, :],
                  preferred_element_type=jnp.float32)
    for c in range(NCH):
        out_ref[c, :, :] = acc[:, c * CK:(c + 1) * CK]


_mm_kernel = pl.pallas_call(
    _mm_body,
    grid=(NRT,),
    in_specs=[
        pl.BlockSpec((RT, D2), lambda m: (m, 0)),
        pl.BlockSpec((RT, D2), lambda m: (m, 0)),
        pl.BlockSpec((RT, 1), lambda m: (m, 0)),
        pl.BlockSpec((D, D2), lambda m: (0, 0)),
    ],
    out_specs=pl.BlockSpec((NCH, RT, CK), lambda m: (0, m, 0)),
    out_shape=jax.ShapeDtypeStruct((NCH, NP, CK), jnp.float32),
)


def _ep_body(h_ref, acc_ref, dis_ref, b_ref, hn_ref):
    dis = dis_ref[...]
    for c in range(NCH):
        sl = slice(c * CK, (c + 1) * CK)
        conv = dis * acc_ref[c, :, :] + b_ref[c, :].reshape(1, CK)
        hn_ref[:, sl] = jnp.maximum(h_ref[:, sl] + conv, 0.0)


_ep_kernel = pl.pallas_call(
    _ep_body,
    grid=(NRT,),
    in_specs=[
        pl.BlockSpec((RT, D2), lambda m: (m, 0)),
        pl.BlockSpec((NCH, RT, CK), lambda m: (0, m, 0)),
        pl.BlockSpec((RT, 1), lambda m: (m, 0)),
        pl.BlockSpec((NCH, CK), lambda m: (0, 0)),
    ],
    out_specs=pl.BlockSpec((RT, D2), lambda m: (m, 0)),
    out_shape=jax.ShapeDtypeStruct((N, D2), jnp.float32),
)


def _pool_body(ha_ref, hb_ref, batch_ref, wr_ref, br_ref, out_ref,
               sums_ref, cnts_ref):
    m = pl.program_id(0)

    @pl.when(m == 0)
    def _init():
        sums_ref[...] = jnp.zeros((NG, D), jnp.float32)
        cnts_ref[...] = jnp.zeros((NG, 1), jnp.float32)

    oh = (lax.broadcasted_iota(jnp.int32, (RT, NG), 1)
          == batch_ref[...]).astype(jnp.float32)
    dn = (((0,), (0,)), ((), ()))
    sums_ref[:, 0:D2] += lax.dot_general(oh, ha_ref[...], dn,
                                         preferred_element_type=jnp.float32)
    sums_ref[:, D2:D] += lax.dot_general(oh, hb_ref[...], dn,
                                         preferred_element_type=jnp.float32)
    cnts_ref[...] += lax.dot_general(oh, jnp.ones((RT, 1), jnp.float32), dn,
                                     preferred_element_type=jnp.float32)

    @pl.when(m == NRT - 1)
    def _fin():
        mean = sums_ref[...] / jnp.maximum(cnts_ref[...], 1.0)
        out_ref[...] = jnp.dot(mean, wr_ref[...],
                               preferred_element_type=jnp.float32) \
            + br_ref[...]


_pool_kernel = pl.pallas_call(
    _pool_body,
    grid=(NRT,),
    in_specs=[
        pl.BlockSpec((RT, D2), lambda m: (m, 0)),
        pl.BlockSpec((RT, D2), lambda m: (m, 0)),
        pl.BlockSpec((RT, 1), lambda m: (m, 0)),
        pl.BlockSpec((D, 2), lambda m: (0, 0)),
        pl.BlockSpec((1, 2), lambda m: (0, 0)),
    ],
    out_specs=pl.BlockSpec((NG, 2), lambda m: (0, 0)),
    out_shape=jax.ShapeDtypeStruct((NG, 2), jnp.float32),
    scratch_shapes=[
        pltpu.VMEM((NG, D), jnp.float32),
        pltpu.VMEM((NG, 1), jnp.float32),
    ],
)


def kernel(x, edge_index, batch, Wc, bc, Wr, br):
    ei = edge_index.astype(jnp.int32)
    src = ei[0].reshape(NSUB, EGRP, EG)
    dst = ei[1].reshape(NSUB, EGRP, EG)
    dst_flat = ei[1]
    batch2d = batch.astype(jnp.int32).reshape(N, 1)
    zeros_dw = jnp.zeros((RPT, DEG_W), jnp.float32)
    ones_deg = jnp.ones((EG, DEG_W), jnp.float32)

    deg_kernel, edge_kernel = _sc_kernels()
    degp = deg_kernel(dst_flat, ones_deg, zeros_dw)
    dis = _pre_kernel(degp)
    ha = x[:, :D2]
    hb = x[:, D2:]
    for i in range(NBLK):
        xws_a = _mm_kernel(ha, hb, dis, Wc[i][:, :D2])
        acc_a = edge_kernel(xws_a.reshape(NCH * NP, CK), src, dst)
        xws_b = _mm_kernel(ha, hb, dis, Wc[i][:, D2:])
        acc_b = edge_kernel(xws_b.reshape(NCH * NP, CK), src, dst)
        ha = _ep_kernel(ha, acc_a, dis, bc[i, :D2].reshape(NCH, CK))
        hb = _ep_kernel(hb, acc_b, dis, bc[i, D2:].reshape(NCH, CK))
    return _pool_kernel(ha, hb, batch2d, Wr, br.reshape(1, 2))


# speedup vs baseline: 1.0080x; 1.0080x over previous
import functools

import jax
import jax.numpy as jnp
from jax import lax
from jax.experimental import pallas as pl
from jax.experimental.pallas import tpu as pltpu
from jax.experimental.pallas import tpu_sc as plsc

NBLK = 4
N = 10000
E = 65536
NG = 64
D = 1024
CK = 128
NCHUNK = D // CK
RT = 1000
NRT = N // RT
NSC = 2
NSUB = 16
EG = 128
EGRP = E // NSUB // EG
DEG_W = 128
DEG_GRP = E // (NSC * NSUB) // EG
NP = 10240
RPT = NP // NSUB


def _deg_body(dst_hbm, ones_hbm, zeros_hbm, out_hbm, ones_v, idx_v, acc_sh):
    core = lax.axis_index("c")
    sub = lax.axis_index("s")
    pltpu.sync_copy(ones_hbm, ones_v)
    pltpu.sync_copy(zeros_hbm, acc_sh.at[pl.ds(sub * RPT, RPT)])
    plsc.subcore_barrier()
    base = (core * NSUB + sub) * (E // (NSC * NSUB))

    def body(g, carry):
        pltpu.sync_copy(dst_hbm.at[pl.ds(base + g * EG, EG)], idx_v)
        pltpu.sync_copy(ones_v, acc_sh.at[idx_v], add=True)
        return carry

    lax.fori_loop(0, DEG_GRP, body, 0)
    plsc.subcore_barrier()
    pltpu.sync_copy(acc_sh.at[pl.ds(sub * RPT, RPT)],
                    out_hbm.at[core, pl.ds(sub * RPT, RPT)])


@functools.cache
def _sc_kernels():
    mesh = plsc.VectorSubcoreMesh(core_axis_name="c", subcore_axis_name="s")
    deg = pl.kernel(
        _deg_body,
        out_type=jax.ShapeDtypeStruct((NSC, NP, DEG_W), jnp.float32),
        mesh=mesh,
        scratch_types=[
            pltpu.VMEM((EG, DEG_W), jnp.float32),
            pltpu.VMEM((EG,), jnp.int32),
            pltpu.VMEM_SHARED((NP, DEG_W), jnp.float32),
        ],
    )
    edge = pl.kernel(
        _edge_body,
        out_type=jax.ShapeDtypeStruct((NCHUNK, NP, CK), jnp.float32),
        mesh=mesh,
        scratch_types=[
            pltpu.VMEM((EG, CK), jnp.float32),
            pltpu.VMEM((EG, CK), jnp.float32),
            pltpu.VMEM((EGRP, EG), jnp.int32),
            pltpu.VMEM((EGRP, EG), jnp.int32),
            pltpu.VMEM((EGRP, EG), jnp.int32),
            pltpu.VMEM_SHARED((NP, CK), jnp.float32),
            pltpu.SemaphoreType.DMA,
            pltpu.SemaphoreType.DMA,
        ],
    )
    return deg, edge


def _edge_body(xws_hbm, src_hbm, dst_hbm, acc_hbm,
               rows0_v, rows1_v, idxs_v, idxo_v, idxd_v, acc_sh, sem0, sem1):
    core = lax.axis_index("c")
    sub = lax.axis_index("s")
    pltpu.sync_copy(src_hbm.at[sub], idxs_v)
    pltpu.sync_copy(dst_hbm.at[sub], idxd_v)

    def chunk(cc, carry):
        chunk_id = core * (NCHUNK // NSC) + cc
        off = chunk_id * NP

        def obody(g, inner):
            for j in range(EG // 16):
                sl = pl.ds(j * 16, 16)
                idxo_v[g, sl] = idxs_v[g, sl] + off
            return inner

        lax.fori_loop(0, EGRP, obody, 0)
        pltpu.sync_copy(xws_hbm.at[pl.ds(off + sub * RPT, RPT)],
                        acc_sh.at[pl.ds(sub * RPT, RPT)])
        plsc.subcore_barrier()

        pltpu.async_copy(xws_hbm.at[idxo_v.at[0]], rows0_v, sem0)

        def g_body(p, inner):
            g0 = 2 * p
            pltpu.async_copy(xws_hbm.at[idxo_v.at[g0 + 1]], rows1_v, sem1)
            pltpu.make_async_copy(xws_hbm.at[idxo_v.at[g0]],
                                  rows0_v, sem0).wait()
            pltpu.sync_copy(rows0_v, acc_sh.at[idxd_v.at[g0]], add=True)

            @pl.when(p + 1 < EGRP // 2)
            def _():
                pltpu.async_copy(xws_hbm.at[idxo_v.at[g0 + 2]], rows0_v, sem0)

            pltpu.make_async_copy(xws_hbm.at[idxo_v.at[g0 + 1]],
                                  rows1_v, sem1).wait()
            pltpu.sync_copy(rows1_v, acc_sh.at[idxd_v.at[g0 + 1]], add=True)
            return inner

        lax.fori_loop(0, EGRP // 2, g_body, 0)
        plsc.subcore_barrier()
        pltpu.sync_copy(acc_sh.at[pl.ds(sub * RPT, RPT)],
                        acc_hbm.at[chunk_id, pl.ds(sub * RPT, RPT)])
        plsc.subcore_barrier()
        return carry

    lax.fori_loop(0, NCHUNK // NSC, chunk, 0)


def _pre_body(degp_ref, dis_ref):
    deg = degp_ref[0, :, 0:1] + degp_ref[1, :, 0:1] + 1.0
    dis_ref[...] = lax.rsqrt(deg)


_pre_kernel = pl.pallas_call(
    _pre_body,
    grid=(NRT,),
    in_specs=[
        pl.BlockSpec((NSC, RT, DEG_W), lambda m: (0, m, 0)),
    ],
    out_specs=pl.BlockSpec((RT, 1), lambda m: (m, 0)),
    out_shape=jax.ShapeDtypeStruct((N, 1), jnp.float32),
)


def _mm_body(h_ref, dis_ref, w_ref, out_ref):
    hs = h_ref[...] * dis_ref[...]
    acc = jnp.dot(hs, w_ref[...], preferred_element_type=jnp.float32)
    for c in range(NCHUNK):
        out_ref[c, :, :] = acc[:, c * CK:(c + 1) * CK]


_mm_kernel = pl.pallas_call(
    _mm_body,
    grid=(NRT,),
    in_specs=[
        pl.BlockSpec((RT, D), lambda m: (m, 0)),
        pl.BlockSpec((RT, 1), lambda m: (m, 0)),
        pl.BlockSpec((D, D), lambda m: (0, 0)),
    ],
    out_specs=pl.BlockSpec((NCHUNK, RT, CK), lambda m: (0, m, 0)),
    out_shape=jax.ShapeDtypeStruct((NCHUNK, NP, CK), jnp.float32),
)


def _ep_body(h_ref, acc_ref, dis_ref, b_ref, hn_ref):
    dis = dis_ref[...]
    for c in range(NCHUNK):
        sl = slice(c * CK, (c + 1) * CK)
        conv = dis * acc_ref[c, :, :] + b_ref[c, :].reshape(1, CK)
        hn_ref[:, sl] = jnp.maximum(h_ref[:, sl] + conv, 0.0)


_ep_kernel = pl.pallas_call(
    _ep_body,
    grid=(NRT,),
    in_specs=[
        pl.BlockSpec((RT, D), lambda m: (m, 0)),
        pl.BlockSpec((NCHUNK, RT, CK), lambda m: (0, m, 0)),
        pl.BlockSpec((RT, 1), lambda m: (m, 0)),
        pl.BlockSpec((NCHUNK, CK), lambda m: (0, 0)),
    ],
    out_specs=pl.BlockSpec((RT, D), lambda m: (m, 0)),
    out_shape=jax.ShapeDtypeStruct((N, D), jnp.float32),
)


def _pool_body(h_ref, batch_ref, wr_ref, br_ref, out_ref, sums_ref, cnts_ref):
    m = pl.program_id(0)

    @pl.when(m == 0)
    def _init():
        sums_ref[...] = jnp.zeros((NG, D), jnp.float32)
        cnts_ref[...] = jnp.zeros((NG, 1), jnp.float32)

    oh = (lax.broadcasted_iota(jnp.int32, (RT, NG), 1)
          == batch_ref[...]).astype(jnp.float32)
    dn = (((0,), (0,)), ((), ()))
    sums_ref[...] += lax.dot_general(oh, h_ref[...], dn,
                                     preferred_element_type=jnp.float32)
    cnts_ref[...] += lax.dot_general(oh, jnp.ones((RT, 1), jnp.float32), dn,
                                     preferred_element_type=jnp.float32)

    @pl.when(m == NRT - 1)
    def _fin():
        mean = sums_ref[...] / jnp.maximum(cnts_ref[...], 1.0)
        out_ref[...] = jnp.dot(mean, wr_ref[...],
                               preferred_element_type=jnp.float32) \
            + br_ref[...]


_pool_kernel = pl.pallas_call(
    _pool_body,
    grid=(NRT,),
    in_specs=[
        pl.BlockSpec((RT, D), lambda m: (m, 0)),
        pl.BlockSpec((RT, 1), lambda m: (m, 0)),
        pl.BlockSpec((D, 2), lambda m: (0, 0)),
        pl.BlockSpec((1, 2), lambda m: (0, 0)),
    ],
    out_specs=pl.BlockSpec((NG, 2), lambda m: (0, 0)),
    out_shape=jax.ShapeDtypeStruct((NG, 2), jnp.float32),
    scratch_shapes=[
        pltpu.VMEM((NG, D), jnp.float32),
        pltpu.VMEM((NG, 1), jnp.float32),
    ],
)


def kernel(x, edge_index, batch, Wc, bc, Wr, br):
    ei = edge_index.astype(jnp.int32)
    src = ei[0].reshape(NSUB, EGRP, EG)
    dst = ei[1].reshape(NSUB, EGRP, EG)
    dst_flat = ei[1]
    batch2d = batch.astype(jnp.int32).reshape(N, 1)
    zeros_dw = jnp.zeros((RPT, DEG_W), jnp.float32)
    ones_deg = jnp.ones((EG, DEG_W), jnp.float32)

    deg_kernel, edge_kernel = _sc_kernels()
    degp = deg_kernel(dst_flat, ones_deg, zeros_dw)
    dis = _pre_kernel(degp)
    h = x
    for i in range(NBLK):
        xws = _mm_kernel(h, dis, Wc[i])
        acc = edge_kernel(xws.reshape(NCHUNK * NP, CK), src, dst)
        h = _ep_kernel(h, acc, dis, bc[i].reshape(NCHUNK, CK))
    return _pool_kernel(h, batch2d, Wr, br.reshape(1, 2))
